# baseline (speedup 1.0000x reference)
import functools

import jax
import jax.numpy as jnp
from jax import lax
from jax.experimental import pallas as pl
from jax.experimental.pallas import tpu as pltpu
from jax.experimental.pallas import tpu_sc as plsc

N = 10000
E = 320000
D = 128
G = 64

NC = 2
NS = 16
K = 64
EPAD = 327680
ROWS = EPAD // K
CPT = ROWS // (NC * NS)
NROW = 10240
NPT = NROW // NS
DW = 16

_mesh = plsc.VectorSubcoreMesh(core_axis_name="c", subcore_axis_name="s")


DK = 128
DCPT = EPAD // DK // (NC * NS)


@functools.partial(
    pl.kernel,
    out_type=jax.ShapeDtypeStruct((NC, NROW, DW), jnp.float32),
    mesh=_mesh,
    scratch_types=[
        pltpu.VMEM_SHARED((NROW, DW), jnp.float32),
        pltpu.VMEM((DK,), jnp.int32),
        pltpu.VMEM((DK, DW), jnp.float32),
    ],
)
def _sc_degree(dst_hbm, ones_hbm, zeros_hbm, out_hbm, deg_sh, idx_v, ones_v):
    c = lax.axis_index("c")
    s = lax.axis_index("s")
    w = c * NS + s
    pltpu.sync_copy(ones_hbm, ones_v)
    pltpu.sync_copy(zeros_hbm, deg_sh.at[pl.ds(s * NPT, NPT)])
    plsc.subcore_barrier()

    @pl.loop(0, DCPT)
    def _chunk(j):
        pltpu.sync_copy(dst_hbm.at[pl.ds((w * DCPT + j) * DK, DK)], idx_v)
        pltpu.sync_copy(ones_v, deg_sh.at[idx_v], add=True)

    plsc.subcore_barrier()
    pltpu.sync_copy(deg_sh.at[pl.ds(s * NPT, NPT)],
                    out_hbm.at[c, pl.ds(s * NPT, NPT)])


NBUF = 5


@functools.partial(
    pl.kernel,
    out_type=jax.ShapeDtypeStruct((NC, NROW, D), jnp.float32),
    mesh=_mesh,
    scratch_types=(
        [pltpu.VMEM_SHARED((NROW, D), jnp.float32)]
        + [pltpu.VMEM((K,), jnp.int32) for _ in range(2 * NBUF)]
        + [pltpu.VMEM((K, D), jnp.float32) for _ in range(NBUF)]
        + [pltpu.SemaphoreType.DMA for _ in range(2 * NBUF)]
    ),
)
def _sc_prop(y_hbm, src_hbm, dst_hbm, zeros_hbm, out_hbm, acc_sh, *bufs):
    si = bufs[0:NBUF]
    di = bufs[NBUF:2 * NBUF]
    rows = bufs[2 * NBUF:3 * NBUF]
    gsem = bufs[3 * NBUF:4 * NBUF]
    ssem = bufs[4 * NBUF:5 * NBUF]
    c = lax.axis_index("c")
    s = lax.axis_index("s")
    w = c * NS + s
    pltpu.sync_copy(zeros_hbm, acc_sh.at[pl.ds(s * NPT, NPT)])
    plsc.subcore_barrier()

    def _load_and_gather(b, chunk):
        e0 = (w * CPT + chunk) * K
        pltpu.sync_copy(src_hbm.at[pl.ds(e0, K)], si[b])
        pltpu.sync_copy(dst_hbm.at[pl.ds(e0, K)], di[b])
        pltpu.async_copy(y_hbm.at[si[b]], rows[b], gsem[b])

    def _wait_gather_scatter(b):
        pltpu.make_async_copy(y_hbm.at[si[b]], rows[b], gsem[b]).wait()
        pltpu.sync_copy(rows[b], acc_sh.at[di[b]], add=True)

    for b in range(NBUF):
        _load_and_gather(b, b)

    @pl.loop(0, CPT - NBUF, step=NBUF)
    def _group(j):
        for b in range(NBUF):
            _wait_gather_scatter(b)
            _load_and_gather(b, j + NBUF + b)

    for b in range(NBUF):
        _wait_gather_scatter(b)

    plsc.subcore_barrier()
    pltpu.sync_copy(acc_sh.at[pl.ds(s * NPT, NPT)],
                    out_hbm.at[c, pl.ds(s * NPT, NPT)])


R = 1000


def _tc_matmul_body(x_ref, w_ref, xw_ref):
    xw_ref[...] = jnp.dot(x_ref[...], w_ref[...],
                          preferred_element_type=jnp.float32)


_tc_matmul = pl.pallas_call(
    _tc_matmul_body,
    grid=(N // R,),
    in_specs=[
        pl.BlockSpec((R, D), lambda i: (i, 0)),
        pl.BlockSpec((D, D), lambda i: (0, 0)),
    ],
    out_specs=pl.BlockSpec((R, D), lambda i: (i, 0)),
    out_shape=jax.ShapeDtypeStruct((N, D), jnp.float32),
)


PAD = EPAD - E


def _pad_mask(i):
    rid = lax.broadcasted_iota(jnp.int32, (R, 1), 0) + i * R
    return (rid < PAD).astype(jnp.float32)


def _tc_scale_body(xw_ref, deg_ref, dis_ref, y_ref):
    deg = deg_ref[...]
    degt = (deg[0, :, 0:1] + deg[1, :, 0:1] + 1.0
            - _pad_mask(pl.program_id(0)))
    dis = lax.rsqrt(degt)
    dis_ref[...] = dis
    y_ref[...] = xw_ref[...] * dis


_tc_scale = pl.pallas_call(
    _tc_scale_body,
    grid=(N // R,),
    in_specs=[
        pl.BlockSpec((R, D), lambda i: (i, 0)),
        pl.BlockSpec((NC, R, DW), lambda i: (0, i, 0)),
    ],
    out_specs=[
        pl.BlockSpec((R, 1), lambda i: (i, 0)),
        pl.BlockSpec((R, D), lambda i: (i, 0)),
    ],
    out_shape=[
        jax.ShapeDtypeStruct((N, 1), jnp.float32),
        jax.ShapeDtypeStruct((N, D), jnp.float32),
    ],
)


def _tc_mid_body(acc_ref, y_ref, dis_ref, b_ref, w_ref, y2_ref):
    a = acc_ref[...]
    y = y_ref[...]
    t = a[0] + a[1] + y * (1.0 - _pad_mask(pl.program_id(0)))
    dis = dis_ref[...]
    h = jnp.maximum(dis * t + b_ref[...], 0.0)
    y2_ref[...] = jnp.dot(h, w_ref[...],
                          preferred_element_type=jnp.float32) * dis


_tc_mid = pl.pallas_call(
    _tc_mid_body,
    grid=(N // R,),
    in_specs=[
        pl.BlockSpec((NC, R, D), lambda i: (0, i, 0)),
        pl.BlockSpec((R, D), lambda i: (i, 0)),
        pl.BlockSpec((R, 1), lambda i: (i, 0)),
        pl.BlockSpec((1, D), lambda i: (0, 0)),
        pl.BlockSpec((D, D), lambda i: (0, 0)),
    ],
    out_specs=pl.BlockSpec((R, D), lambda i: (i, 0)),
    out_shape=jax.ShapeDtypeStruct((N, D), jnp.float32),
)


def _tc_final_body(acc_ref, y_ref, dis_ref, b_ref, batch_ref, fcw_ref,
                   fcb_ref, out_ref, ps_ref, cnt_ref):
    i = pl.program_id(0)

    @pl.when(i == 0)
    def _():
        ps_ref[...] = jnp.zeros_like(ps_ref)
        cnt_ref[...] = jnp.zeros_like(cnt_ref)

    a = acc_ref[...]
    y = y_ref[...]
    t = a[0] + a[1] + y * (1.0 - _pad_mask(i))
    h = jnp.maximum(dis_ref[...] * t + b_ref[...], 0.0)
    gids = lax.broadcasted_iota(jnp.int32, (G, R), 0)
    m = (gids == batch_ref[...].reshape(1, R)).astype(jnp.float32)
    ps_ref[...] += jnp.dot(m, h, preferred_element_type=jnp.float32)
    cnt_ref[...] += jnp.sum(m, axis=1, keepdims=True)

    @pl.when(i == N // R - 1)
    def _():
        pooled = ps_ref[...] / jnp.maximum(cnt_ref[...], 1.0)
        out_ref[...] = jnp.dot(pooled, fcw_ref[...],
                               preferred_element_type=jnp.float32) + fcb_ref[...]


_tc_final = pl.pallas_call(
    _tc_final_body,
    grid=(N // R,),
    in_specs=[
        pl.BlockSpec((NC, R, D), lambda i: (0, i, 0)),
        pl.BlockSpec((R, D), lambda i: (i, 0)),
        pl.BlockSpec((R, 1), lambda i: (i, 0)),
        pl.BlockSpec((1, D), lambda i: (0, 0)),
        pl.BlockSpec((1, 1, R), lambda i: (i, 0, 0)),
        pl.BlockSpec((D, D), lambda i: (0, 0)),
        pl.BlockSpec((1, D), lambda i: (0, 0)),
    ],
    out_specs=pl.BlockSpec((G, D), lambda i: (0, 0)),
    out_shape=jax.ShapeDtypeStruct((G, D), jnp.float32),
    scratch_shapes=[
        pltpu.VMEM((G, D), jnp.float32),
        pltpu.VMEM((G, 1), jnp.float32),
    ],
)


def kernel(x, edge_index, batch, W1, b1, W2, b2, fc_W, fc_b):
    pad = EPAD - E
    pad_ids = jnp.arange(pad, dtype=jnp.int32)
    src_p = jnp.concatenate([edge_index[0], pad_ids])
    dst_p = jnp.concatenate([edge_index[1], pad_ids])
    zeros_d = jnp.zeros((NPT, D), jnp.float32)
    zeros_w = jnp.zeros((NPT, DW), jnp.float32)
    ones_w = jnp.ones((DK, DW), jnp.float32)

    xw = _tc_matmul(x, W1)
    deg = _sc_degree(dst_p, ones_w, zeros_w)
    dis, y1 = _tc_scale(xw, deg)
    acc1 = _sc_prop(y1, src_p, dst_p, zeros_d)
    y2 = _tc_mid(acc1, y1, dis, b1.reshape(1, D), W2)
    acc2 = _sc_prop(y2, src_p, dst_p, zeros_d)
    out = _tc_final(acc2, y2, dis, b2.reshape(1, D), batch.reshape(N // R, 1, R),
                    fc_W, fc_b.reshape(1, D))
    return out


# speedup vs baseline: 1.0011x; 1.0011x over previous
import functools

import jax
import jax.numpy as jnp
from jax import lax
from jax.experimental import pallas as pl
from jax.experimental.pallas import tpu as pltpu
from jax.experimental.pallas import tpu_sc as plsc

N = 10000
E = 320000
D = 128
G = 64

NC = 2
NS = 16
K = 64
EPAD = 327680
ROWS = EPAD // K
CPT = ROWS // (NC * NS)
NROW = 10240
NPT = NROW // NS
DW = 16

_mesh = plsc.VectorSubcoreMesh(core_axis_name="c", subcore_axis_name="s")


DK = 128
DCPT = EPAD // DK // (NC * NS)


@functools.partial(
    pl.kernel,
    out_type=jax.ShapeDtypeStruct((NC, NROW, DW), jnp.float32),
    mesh=_mesh,
    scratch_types=[
        pltpu.VMEM_SHARED((NROW, DW), jnp.float32),
        pltpu.VMEM((DK,), jnp.int32),
        pltpu.VMEM((DK, DW), jnp.float32),
    ],
)
def _sc_degree(dst_hbm, ones_hbm, zeros_hbm, out_hbm, deg_sh, idx_v, ones_v):
    c = lax.axis_index("c")
    s = lax.axis_index("s")
    w = c * NS + s
    pltpu.sync_copy(ones_hbm, ones_v)
    pltpu.sync_copy(zeros_hbm, deg_sh.at[pl.ds(s * NPT, NPT)])
    plsc.subcore_barrier()

    @pl.loop(0, DCPT)
    def _chunk(j):
        pltpu.sync_copy(dst_hbm.at[pl.ds((w * DCPT + j) * DK, DK)], idx_v)
        pltpu.sync_copy(ones_v, deg_sh.at[idx_v], add=True)

    plsc.subcore_barrier()
    pltpu.sync_copy(deg_sh.at[pl.ds(s * NPT, NPT)],
                    out_hbm.at[c, pl.ds(s * NPT, NPT)])


NBUF = 4


@functools.partial(
    pl.kernel,
    out_type=jax.ShapeDtypeStruct((NC, NROW, D), jnp.float32),
    mesh=_mesh,
    scratch_types=(
        [pltpu.VMEM_SHARED((NROW, D), jnp.float32)]
        + [pltpu.VMEM((K,), jnp.int32) for _ in range(2 * NBUF)]
        + [pltpu.VMEM((K, D), jnp.float32) for _ in range(NBUF)]
        + [pltpu.SemaphoreType.DMA for _ in range(2 * NBUF)]
    ),
)
def _sc_prop(y_hbm, src_hbm, dst_hbm, zeros_hbm, out_hbm, acc_sh, *bufs):
    si = bufs[0:NBUF]
    di = bufs[NBUF:2 * NBUF]
    rows = bufs[2 * NBUF:3 * NBUF]
    gsem = bufs[3 * NBUF:4 * NBUF]
    ssem = bufs[4 * NBUF:5 * NBUF]
    c = lax.axis_index("c")
    s = lax.axis_index("s")
    w = c * NS + s
    pltpu.sync_copy(zeros_hbm, acc_sh.at[pl.ds(s * NPT, NPT)])
    plsc.subcore_barrier()

    def _load_and_gather(b, chunk):
        e0 = (w * CPT + chunk) * K
        pltpu.sync_copy(src_hbm.at[pl.ds(e0, K)], si[b])
        pltpu.sync_copy(dst_hbm.at[pl.ds(e0, K)], di[b])
        pltpu.async_copy(y_hbm.at[si[b]], rows[b], gsem[b])

    def _wait_gather_scatter(b):
        pltpu.make_async_copy(y_hbm.at[si[b]], rows[b], gsem[b]).wait()
        pltpu.sync_copy(rows[b], acc_sh.at[di[b]], add=True)

    for b in range(NBUF):
        _load_and_gather(b, b)

    @pl.loop(0, CPT - NBUF, step=NBUF)
    def _group(j):
        for b in range(NBUF):
            _wait_gather_scatter(b)
            _load_and_gather(b, j + NBUF + b)

    for b in range(NBUF):
        _wait_gather_scatter(b)

    plsc.subcore_barrier()
    pltpu.sync_copy(acc_sh.at[pl.ds(s * NPT, NPT)],
                    out_hbm.at[c, pl.ds(s * NPT, NPT)])


R = 1000


def _tc_matmul_body(x_ref, w_ref, xw_ref):
    xw_ref[...] = jnp.dot(x_ref[...], w_ref[...],
                          preferred_element_type=jnp.float32)


_tc_matmul = pl.pallas_call(
    _tc_matmul_body,
    grid=(N // R,),
    in_specs=[
        pl.BlockSpec((R, D), lambda i: (i, 0)),
        pl.BlockSpec((D, D), lambda i: (0, 0)),
    ],
    out_specs=pl.BlockSpec((R, D), lambda i: (i, 0)),
    out_shape=jax.ShapeDtypeStruct((N, D), jnp.float32),
)


PAD = EPAD - E


def _pad_mask(i):
    rid = lax.broadcasted_iota(jnp.int32, (R, 1), 0) + i * R
    return (rid < PAD).astype(jnp.float32)


def _tc_scale_body(xw_ref, deg_ref, dis_ref, y_ref):
    deg = deg_ref[...]
    degt = (deg[0, :, 0:1] + deg[1, :, 0:1] + 1.0
            - _pad_mask(pl.program_id(0)))
    dis = lax.rsqrt(degt)
    dis_ref[...] = dis
    y_ref[...] = xw_ref[...] * dis


_tc_scale = pl.pallas_call(
    _tc_scale_body,
    grid=(N // R,),
    in_specs=[
        pl.BlockSpec((R, D), lambda i: (i, 0)),
        pl.BlockSpec((NC, R, DW), lambda i: (0, i, 0)),
    ],
    out_specs=[
        pl.BlockSpec((R, 1), lambda i: (i, 0)),
        pl.BlockSpec((R, D), lambda i: (i, 0)),
    ],
    out_shape=[
        jax.ShapeDtypeStruct((N, 1), jnp.float32),
        jax.ShapeDtypeStruct((N, D), jnp.float32),
    ],
)


def _tc_mid_body(acc_ref, y_ref, dis_ref, b_ref, w_ref, y2_ref):
    a = acc_ref[...]
    y = y_ref[...]
    t = a[0] + a[1] + y * (1.0 - _pad_mask(pl.program_id(0)))
    dis = dis_ref[...]
    h = jnp.maximum(dis * t + b_ref[...], 0.0)
    y2_ref[...] = jnp.dot(h, w_ref[...],
                          preferred_element_type=jnp.float32) * dis


_tc_mid = pl.pallas_call(
    _tc_mid_body,
    grid=(N // R,),
    in_specs=[
        pl.BlockSpec((NC, R, D), lambda i: (0, i, 0)),
        pl.BlockSpec((R, D), lambda i: (i, 0)),
        pl.BlockSpec((R, 1), lambda i: (i, 0)),
        pl.BlockSpec((1, D), lambda i: (0, 0)),
        pl.BlockSpec((D, D), lambda i: (0, 0)),
    ],
    out_specs=pl.BlockSpec((R, D), lambda i: (i, 0)),
    out_shape=jax.ShapeDtypeStruct((N, D), jnp.float32),
)


def _tc_final_body(acc_ref, y_ref, dis_ref, b_ref, batch_ref, fcw_ref,
                   fcb_ref, out_ref, ps_ref, cnt_ref):
    i = pl.program_id(0)

    @pl.when(i == 0)
    def _():
        ps_ref[...] = jnp.zeros_like(ps_ref)
        cnt_ref[...] = jnp.zeros_like(cnt_ref)

    a = acc_ref[...]
    y = y_ref[...]
    t = a[0] + a[1] + y * (1.0 - _pad_mask(i))
    h = jnp.maximum(dis_ref[...] * t + b_ref[...], 0.0)
    gids = lax.broadcasted_iota(jnp.int32, (G, R), 0)
    m = (gids == batch_ref[...].reshape(1, R)).astype(jnp.float32)
    ps_ref[...] += jnp.dot(m, h, preferred_element_type=jnp.float32)
    cnt_ref[...] += jnp.sum(m, axis=1, keepdims=True)

    @pl.when(i == N // R - 1)
    def _():
        pooled = ps_ref[...] / jnp.maximum(cnt_ref[...], 1.0)
        out_ref[...] = jnp.dot(pooled, fcw_ref[...],
                               preferred_element_type=jnp.float32) + fcb_ref[...]


_tc_final = pl.pallas_call(
    _tc_final_body,
    grid=(N // R,),
    in_specs=[
        pl.BlockSpec((NC, R, D), lambda i: (0, i, 0)),
        pl.BlockSpec((R, D), lambda i: (i, 0)),
        pl.BlockSpec((R, 1), lambda i: (i, 0)),
        pl.BlockSpec((1, D), lambda i: (0, 0)),
        pl.BlockSpec((1, 1, R), lambda i: (i, 0, 0)),
        pl.BlockSpec((D, D), lambda i: (0, 0)),
        pl.BlockSpec((1, D), lambda i: (0, 0)),
    ],
    out_specs=pl.BlockSpec((G, D), lambda i: (0, 0)),
    out_shape=jax.ShapeDtypeStruct((G, D), jnp.float32),
    scratch_shapes=[
        pltpu.VMEM((G, D), jnp.float32),
        pltpu.VMEM((G, 1), jnp.float32),
    ],
)


def kernel(x, edge_index, batch, W1, b1, W2, b2, fc_W, fc_b):
    pad = EPAD - E
    pad_ids = jnp.arange(pad, dtype=jnp.int32)
    src_p = jnp.concatenate([edge_index[0], pad_ids])
    dst_p = jnp.concatenate([edge_index[1], pad_ids])
    zeros_d = jnp.zeros((NPT, D), jnp.float32)
    zeros_w = jnp.zeros((NPT, DW), jnp.float32)
    ones_w = jnp.ones((DK, DW), jnp.float32)

    xw = _tc_matmul(x, W1)
    deg = _sc_degree(dst_p, ones_w, zeros_w)
    dis, y1 = _tc_scale(xw, deg)
    acc1 = _sc_prop(y1, src_p, dst_p, zeros_d)
    y2 = _tc_mid(acc1, y1, dis, b1.reshape(1, D), W2)
    acc2 = _sc_prop(y2, src_p, dst_p, zeros_d)
    out = _tc_final(acc2, y2, dis, b2.reshape(1, D), batch.reshape(N // R, 1, R),
                    fc_W, fc_b.reshape(1, D))
    return out
